# speedup vs baseline: 6.9065x; 6.9065x over previous
import functools

import jax
import jax.numpy as jnp
from jax import lax
from jax.experimental import pallas as pl
from jax.experimental.pallas import tpu as pltpu
from jax.experimental.pallas import tpu_sc as plsc

NUM_REL = 3
N = 10000
E = 320000
D = 128

NUM_CORES = 2
NUM_SUBCORES = 16
NW = NUM_CORES * NUM_SUBCORES
CH = 128
NCHUNK = -(-E // (NW * CH))
EPAD = NW * CH * NCHUNK
ACC_ROWS = 10240
ROWS_PER_TILE = ACC_ROWS // NUM_SUBCORES
MM_BLK = 2000
NB = N // MM_BLK


def _mm_body(x_ref, w_ref, o_ref):
    o_ref[...] = jnp.dot(x_ref[...], w_ref[0],
                         preferred_element_type=jnp.float32)


def _project(x, w):
    return pl.pallas_call(
        _mm_body,
        grid=(NUM_REL, NB),
        in_specs=[
            pl.BlockSpec((MM_BLK, D), lambda r, n: (n, 0)),
            pl.BlockSpec((1, D, D), lambda r, n: (r, 0, 0)),
        ],
        out_specs=pl.BlockSpec((MM_BLK, D), lambda r, n: (r * NB + n, 0)),
        out_shape=jax.ShapeDtypeStruct((NUM_REL * N, D), jnp.float32),
    )(x, w)


def _relu_mm_body(p_ref, w_ref, o_ref):
    h = jnp.maximum(p_ref[0] + p_ref[1], 0.0)
    o_ref[...] = jnp.dot(h, w_ref[0], preferred_element_type=jnp.float32)


def _relu_project(p, w):
    return pl.pallas_call(
        _relu_mm_body,
        grid=(NUM_REL, NB),
        in_specs=[
            pl.BlockSpec((NUM_CORES, MM_BLK, D), lambda r, n: (0, n, 0)),
            pl.BlockSpec((1, D, D), lambda r, n: (r, 0, 0)),
        ],
        out_specs=pl.BlockSpec((MM_BLK, D), lambda r, n: (r * NB + n, 0)),
        out_shape=jax.ShapeDtypeStruct((NUM_REL * N, D), jnp.float32),
    )(p, w)


def _logsoftmax_body(p_ref, o_ref):
    x = p_ref[0] + p_ref[1]
    m = jnp.max(x, axis=1, keepdims=True)
    ex = jnp.exp(x - m)
    lse = jnp.log(jnp.sum(ex, axis=1, keepdims=True)) + m
    o_ref[...] = x - lse


def _sum_logsoftmax(p):
    return pl.pallas_call(
        _logsoftmax_body,
        grid=(NB,),
        in_specs=[pl.BlockSpec((NUM_CORES, MM_BLK, D), lambda n: (0, n, 0))],
        out_specs=pl.BlockSpec((MM_BLK, D), lambda n: (n, 0)),
        out_shape=jax.ShapeDtypeStruct((N, D), jnp.float32),
    )(p)


_SC_MESH = plsc.VectorSubcoreMesh(core_axis_name="c", subcore_axis_name="s")


@functools.partial(
    pl.kernel,
    out_type=jax.ShapeDtypeStruct((NUM_CORES, ACC_ROWS, D), jnp.float32),
    mesh=_SC_MESH,
    scratch_types=[
        pltpu.VMEM((CH,), jnp.int32),
        pltpu.VMEM((CH,), jnp.int32),
        pltpu.VMEM((CH,), jnp.int32),
        pltpu.VMEM((CH,), jnp.int32),
        pltpu.VMEM((CH, D), jnp.float32),
        pltpu.VMEM_SHARED((ACC_ROWS, D), jnp.float32),
        pltpu.SemaphoreType.DMA,
    ],
)
def _sc_aggregate(y_hbm, src_hbm, typ_hbm, dst_hbm, zeros_hbm, out_hbm,
                  s_v, t_v, d_v, i_v, rows_v, acc, sem):
    c = lax.axis_index("c")
    s = lax.axis_index("s")
    wid = c * NUM_SUBCORES + s

    pltpu.sync_copy(zeros_hbm, acc.at[pl.ds(s * ROWS_PER_TILE, ROWS_PER_TILE)])
    plsc.subcore_barrier()

    def chunk(j, carry):
        base = (wid * NCHUNK + j) * CH
        pltpu.sync_copy(src_hbm.at[pl.ds(base, CH)], s_v)
        pltpu.sync_copy(typ_hbm.at[pl.ds(base, CH)], t_v)
        pltpu.sync_copy(dst_hbm.at[pl.ds(base, CH)], d_v)
        for i in range(CH // 16):
            sl = pl.ds(i * 16, 16)
            i_v[sl] = t_v[sl] * N + s_v[sl]
        pltpu.async_copy(y_hbm.at[i_v], rows_v, sem).wait()
        pltpu.sync_copy(rows_v, acc.at[d_v], add=True)
        return carry

    lax.fori_loop(0, NCHUNK, chunk, 0)
    plsc.subcore_barrier()

    sl = pl.ds(s * ROWS_PER_TILE, ROWS_PER_TILE)
    pltpu.sync_copy(acc.at[sl], out_hbm.at[c, sl])


@jax.jit
def kernel(X, edge_index, edge_type, W1, W2):
    pad = EPAD - E
    src = jnp.concatenate([edge_index[0], jnp.zeros((pad,), jnp.int32)])
    typ = jnp.concatenate([edge_type, jnp.zeros((pad,), jnp.int32)])
    dst = jnp.concatenate([edge_index[1], jnp.full((pad,), N, jnp.int32)])
    zeros = jnp.zeros((ROWS_PER_TILE, D), jnp.float32)

    y1 = _project(X, W1)
    p1 = _sc_aggregate(y1, src, typ, dst, zeros)
    y2 = _relu_project(p1, W2)
    p2 = _sc_aggregate(y2, src, typ, dst, zeros)
    return _sum_logsoftmax(p2)
